# Initial kernel scaffold; baseline (speedup 1.0000x reference)
import jax
import jax.numpy as jnp
from jax.experimental import pallas as pl


def kernel(input, weight):
    raise NotImplementedError("write your pallas kernel here")


# speedup vs baseline: 4.0818x; 4.0818x over previous
import functools

import jax
import jax.numpy as jnp
from jax import lax
from jax.experimental import pallas as pl
from jax.experimental.pallas import tpu as pltpu
from jax.experimental.pallas import tpu_sc as plsc

_NC = 2
_NS = 16
_NW = _NC * _NS
_GRP = 128


def _make_gather(n_grp, D):
    grp_per_w = n_grp // _NW
    mesh = plsc.VectorSubcoreMesh(core_axis_name="c", subcore_axis_name="s")

    @functools.partial(
        pl.kernel,
        mesh=mesh,
        out_type=jax.ShapeDtypeStruct((n_grp * _GRP, D), jnp.float32),
        scratch_types=[
            pltpu.VMEM((grp_per_w * _GRP,), jnp.int32),
            pltpu.VMEM((_GRP, D), jnp.float32),
            pltpu.SemaphoreType.DMA,
        ],
        compiler_params=pltpu.CompilerParams(use_tc_tiling_on_sc=False),
    )
    def k(idx_hbm, table_hbm, out_hbm, idx_v, rows_v, sem):
        wid = lax.axis_index("s") * _NC + lax.axis_index("c")
        gbase = wid * grp_per_w
        pltpu.sync_copy(idx_hbm.at[pl.ds(gbase * _GRP, grp_per_w * _GRP)], idx_v)

        def body(j, carry):
            pltpu.async_copy(
                table_hbm.at[idx_v.at[pl.ds(j * _GRP, _GRP)]], rows_v, sem
            ).wait()
            pltpu.sync_copy(rows_v, out_hbm.at[pl.ds((gbase + j) * _GRP, _GRP)])
            return carry

        lax.fori_loop(0, grp_per_w, body, 0)

    return k


def kernel(input, weight):
    S0, S1 = input.shape
    B = S0 * S1
    D = weight.shape[1]
    idx = input.reshape(B).astype(jnp.int32)
    out = _make_gather(B // _GRP, D)(idx, weight)
    return out.reshape(S0, S1, D)


# speedup vs baseline: 4.6171x; 1.1312x over previous
import functools

import jax
import jax.numpy as jnp
from jax import lax
from jax.experimental import pallas as pl
from jax.experimental.pallas import tpu as pltpu
from jax.experimental.pallas import tpu_sc as plsc

_NC = 2
_NS = 16
_NW = _NC * _NS
_GRP = 128
_K = 5


def _make_gather(n_grp, D):
    grp_per_w = n_grp // _NW
    n_chunk = grp_per_w // _K
    mesh = plsc.VectorSubcoreMesh(core_axis_name="c", subcore_axis_name="s")

    @functools.partial(
        pl.kernel,
        mesh=mesh,
        out_type=jax.ShapeDtypeStruct((n_grp * _GRP, D), jnp.float32),
        scratch_types=[
            pltpu.VMEM((grp_per_w * _GRP,), jnp.int32),
            pltpu.VMEM((_K * _GRP, D), jnp.float32),
            pltpu.VMEM((_K * _GRP, D), jnp.float32),
            pltpu.SemaphoreType.DMA,
            pltpu.SemaphoreType.DMA,
            pltpu.SemaphoreType.DMA,
            pltpu.SemaphoreType.DMA,
        ],
        compiler_params=pltpu.CompilerParams(use_tc_tiling_on_sc=False),
    )
    def k(idx_hbm, table_hbm, out_hbm, idx_v, buf_a, buf_b, sga, sgb, ssa, ssb):
        wid = lax.axis_index("s") * _NC + lax.axis_index("c")
        gbase = wid * grp_per_w
        pltpu.sync_copy(idx_hbm.at[pl.ds(gbase * _GRP, grp_per_w * _GRP)], idx_v)

        def g_start(c, buf, sem):
            for b in range(_K):
                pltpu.make_async_copy(
                    table_hbm.at[idx_v.at[pl.ds((c * _K + b) * _GRP, _GRP)]],
                    buf.at[pl.ds(b * _GRP, _GRP)],
                    sem,
                ).start()

        def g_wait(buf, sem):
            for b in range(_K):
                pltpu.make_async_copy(
                    table_hbm.at[idx_v.at[pl.ds(0, _GRP)]],
                    buf.at[pl.ds(b * _GRP, _GRP)],
                    sem,
                ).wait()

        def s_start(c, buf, sem):
            for b in range(_K):
                pltpu.make_async_copy(
                    buf.at[pl.ds(b * _GRP, _GRP)],
                    out_hbm.at[pl.ds((gbase + c * _K + b) * _GRP, _GRP)],
                    sem,
                ).start()

        def s_wait(buf, sem):
            for b in range(_K):
                pltpu.make_async_copy(
                    buf.at[pl.ds(b * _GRP, _GRP)],
                    out_hbm.at[pl.ds(gbase * _GRP, _GRP)],
                    sem,
                ).wait()

        g_start(0, buf_a, sga)
        g_wait(buf_a, sga)
        g_start(1, buf_b, sgb)
        s_start(0, buf_a, ssa)

        def pair(p, carry):
            c1 = 2 * p + 1
            g_wait(buf_b, sgb)
            s_wait(buf_a, ssa)
            g_start(c1 + 1, buf_a, sga)
            s_start(c1, buf_b, ssb)
            c2 = 2 * p + 2
            g_wait(buf_a, sga)
            s_wait(buf_b, ssb)
            g_start(c2 + 1, buf_b, sgb)
            s_start(c2, buf_a, ssa)
            return carry

        lax.fori_loop(0, n_chunk // 2 - 1, pair, 0)

        g_wait(buf_b, sgb)
        s_wait(buf_a, ssa)
        s_start(n_chunk - 1, buf_b, ssb)
        s_wait(buf_b, ssb)

    return k


def kernel(input, weight):
    S0, S1 = input.shape
    B = S0 * S1
    D = weight.shape[1]
    idx = input.reshape(B).astype(jnp.int32)
    out = _make_gather(B // _GRP, D)(idx, weight)
    return out.reshape(S0, S1, D)
